# baseline (speedup 1.0000x reference)
import functools

import jax
import jax.numpy as jnp
from jax import lax
from jax.experimental import pallas as pl
from jax.experimental.pallas import tpu as pltpu
from jax.experimental.pallas import tpu_sc as plsc

LANES = 16
NBUF = 5
DEPTH = 2


@functools.lru_cache(maxsize=None)
def _build(N, V, D, C):
    info = plsc.get_sparse_core_info()
    NC, NS = info.num_cores, info.num_subcores
    NW = NC * NS
    per_w = N // NW
    n_chunks = per_w // C
    n_groups = n_chunks // NBUF
    n_col = D // LANES
    mesh = plsc.VectorSubcoreMesh(core_axis_name="c", subcore_axis_name="s")

    scratch = (
        [pltpu.VMEM((C,), jnp.int32) for _ in range(NBUF)]
        + [pltpu.VMEM((C,), jnp.float32) for _ in range(NBUF)]
        + [pltpu.VMEM((C, D), jnp.float32) for _ in range(NBUF)]
        + [pltpu.VMEM((D,), jnp.float32), pltpu.VMEM((D,), jnp.float32)]
        + [pltpu.VMEM_SHARED((V, D), jnp.float32)]
        + [pltpu.SemaphoreType.DMA for _ in range(2 * NBUF)]
    )

    @functools.partial(
        pl.kernel,
        mesh=mesh,
        out_type=jax.ShapeDtypeStruct((N, D), jnp.float32),
        compiler_params=pltpu.CompilerParams(needs_layout_passes=False),
        scratch_types=scratch,
    )
    def k(x_hbm, y_hbm, emb_hbm, w_hbm, b_hbm, out_hbm, *s):
        idx = s[0:NBUF]
        yv = s[NBUF:2 * NBUF]
        rows = s[2 * NBUF:3 * NBUF]
        w_v, b_v = s[3 * NBUF], s[3 * NBUF + 1]
        emb_sp = s[3 * NBUF + 2]
        gsem = s[3 * NBUF + 3:4 * NBUF + 3]
        osem = s[4 * NBUF + 3:5 * NBUF + 3]

        sid = lax.axis_index("s")
        wid = sid * NC + lax.axis_index("c")
        w0 = wid * per_w

        @pl.when(sid == 0)
        def _():
            pltpu.sync_copy(emb_hbm, emb_sp)

        pltpu.sync_copy(w_hbm, w_v)
        pltpu.sync_copy(b_hbm, b_v)
        w_regs = [w_v[pl.ds(j * LANES, LANES)] for j in range(n_col)]
        b_regs = [b_v[pl.ds(j * LANES, LANES)] for j in range(n_col)]
        plsc.subcore_barrier()

        def gdesc(ci, b):
            return pltpu.make_async_copy(emb_sp.at[idx[b]], rows[b], gsem[b])

        def fetch(ci, b):
            base = w0 + ci * C
            pltpu.sync_copy(x_hbm.at[pl.ds(base, C)], idx[b])
            pltpu.sync_copy(y_hbm.at[pl.ds(base, C)], yv[b])
            gdesc(ci, b).start()

        def odesc(ci, b):
            return pltpu.make_async_copy(
                rows[b], out_hbm.at[pl.ds(w0 + ci * C, C)], osem[b]
            )

        for b in range(DEPTH):
            fetch(b, b)

        def group(g, carry):
            for b in range(NBUF):
                ci = g * NBUF + b
                p = ci + DEPTH
                pb = (b + DEPTH) % NBUF

                @pl.when((p >= NBUF) & (p < n_chunks))
                def _():
                    odesc(p - NBUF, pb).wait()

                @pl.when(p < n_chunks)
                def _():
                    fetch(p, pb)

                gdesc(ci, b).wait()

                def blk_body(r16, acc):
                    r0 = r16 * LANES
                    y16 = yv[b][pl.ds(r0, LANES)]
                    for kk in range(LANES):
                        ysplat = jnp.take_along_axis(
                            y16,
                            jnp.full((LANES,), kk, jnp.int32),
                            axis=0,
                            mode="promise_in_bounds",
                        )
                        for j in range(n_col):
                            plsc.addupdate(
                                rows[b].at[r0 + kk, pl.ds(j * LANES, LANES)],
                                ysplat * w_regs[j] + b_regs[j],
                            )
                    return acc

                lax.fori_loop(0, C // LANES, blk_body, 0, unroll=False)
                odesc(ci, b).start()
            return carry

        lax.fori_loop(0, n_groups, group, 0, unroll=False)
        for b in range(NBUF):
            odesc(n_chunks - NBUF + b, b).wait()

    return k


def kernel(x, y, embedding, W_sig, b_sig):
    B, L = x.shape
    V, D = embedding.shape
    N = B * L
    xf = x.reshape(N)
    yf = y.reshape(N)
    w = W_sig.reshape(D)
    out = _build(N, V, D, 128)(xf, yf, embedding, w, b_sig)
    return out.reshape(B, L, D)


# speedup vs baseline: 1.5281x; 1.5281x over previous
import functools

import jax
import jax.numpy as jnp
from jax import lax
from jax.experimental import pallas as pl
from jax.experimental.pallas import tpu as pltpu
from jax.experimental.pallas import tpu_sc as plsc

LANES = 16
NBUF = 5
DEPTH = 2


@functools.lru_cache(maxsize=None)
def _build(N, V, D, C):
    info = plsc.get_sparse_core_info()
    NC, NS = info.num_cores, info.num_subcores
    NW = NC * NS
    per_w = N // NW
    n_chunks = per_w // C
    n_groups = n_chunks // NBUF
    n_col = D // LANES
    mesh = plsc.VectorSubcoreMesh(core_axis_name="c", subcore_axis_name="s")

    scratch = (
        [pltpu.VMEM((per_w,), jnp.int32)]
        + [pltpu.VMEM((C,), jnp.float32) for _ in range(NBUF)]
        + [pltpu.VMEM((C, D), jnp.float32) for _ in range(NBUF)]
        + [pltpu.VMEM((D,), jnp.float32), pltpu.VMEM((D,), jnp.float32)]
        + [pltpu.VMEM_SHARED((V, D), jnp.float32)]
        + [pltpu.SemaphoreType.DMA for _ in range(3 * NBUF)]
    )

    @functools.partial(
        pl.kernel,
        mesh=mesh,
        out_type=jax.ShapeDtypeStruct((N, D), jnp.float32),
        compiler_params=pltpu.CompilerParams(needs_layout_passes=False),
        scratch_types=scratch,
    )
    def k(x_hbm, y_hbm, emb_hbm, w_hbm, b_hbm, out_hbm, *s):
        idx_all = s[0]
        yv = s[1:1 + NBUF]
        rows = s[1 + NBUF:1 + 2 * NBUF]
        w_v, b_v = s[1 + 2 * NBUF], s[2 + 2 * NBUF]
        emb_sp = s[3 + 2 * NBUF]
        gsem = s[4 + 2 * NBUF:4 + 3 * NBUF]
        osem = s[4 + 3 * NBUF:4 + 4 * NBUF]
        ysem = s[4 + 4 * NBUF:4 + 5 * NBUF]

        sid = lax.axis_index("s")
        wid = sid * NC + lax.axis_index("c")
        w0 = wid * per_w

        @pl.when(sid == 0)
        def _():
            pltpu.sync_copy(emb_hbm, emb_sp)

        pltpu.sync_copy(x_hbm.at[pl.ds(w0, per_w)], idx_all)
        pltpu.sync_copy(w_hbm, w_v)
        pltpu.sync_copy(b_hbm, b_v)
        w_regs = [w_v[pl.ds(j * LANES, LANES)] for j in range(n_col)]
        b_regs = [b_v[pl.ds(j * LANES, LANES)] for j in range(n_col)]
        plsc.subcore_barrier()

        def gdesc(ci, b):
            return pltpu.make_async_copy(
                emb_sp.at[idx_all.at[pl.ds(ci * C, C)]], rows[b], gsem[b]
            )

        def ydesc(ci, b):
            return pltpu.make_async_copy(
                y_hbm.at[pl.ds(w0 + ci * C, C)], yv[b], ysem[b]
            )

        def odesc(ci, b):
            return pltpu.make_async_copy(
                rows[b], out_hbm.at[pl.ds(w0 + ci * C, C)], osem[b]
            )

        for b in range(DEPTH):
            ydesc(b, b).start()
            gdesc(b, b).start()

        def group(g, carry):
            for b in range(NBUF):
                ci = g * NBUF + b
                p = ci + DEPTH
                pb = (b + DEPTH) % NBUF

                @pl.when((p >= NBUF) & (p < n_chunks))
                def _():
                    odesc(p - NBUF, pb).wait()

                @pl.when(p < n_chunks)
                def _():
                    ydesc(p, pb).start()
                    gdesc(p, pb).start()

                gdesc(ci, b).wait()
                ydesc(ci, b).wait()

                def blk_body(r16, acc):
                    r0 = r16 * LANES
                    y16 = yv[b][pl.ds(r0, LANES)]
                    for kk in range(LANES):
                        ysplat = jnp.take_along_axis(
                            y16,
                            jnp.full((LANES,), kk, jnp.int32),
                            axis=0,
                            mode="promise_in_bounds",
                        )
                        for j in range(n_col):
                            plsc.addupdate(
                                rows[b].at[r0 + kk, pl.ds(j * LANES, LANES)],
                                ysplat * w_regs[j] + b_regs[j],
                            )
                    return acc

                lax.fori_loop(0, C // LANES, blk_body, 0, unroll=False)
                odesc(ci, b).start()
            return carry

        lax.fori_loop(0, n_groups, group, 0, unroll=False)
        for b in range(NBUF):
            odesc(n_chunks - NBUF + b, b).wait()

    return k


def kernel(x, y, embedding, W_sig, b_sig):
    B, L = x.shape
    V, D = embedding.shape
    N = B * L
    xf = x.reshape(N)
    yf = y.reshape(N)
    w = W_sig.reshape(D)
    out = _build(N, V, D, 128)(xf, yf, embedding, w, b_sig)
    return out.reshape(B, L, D)


# speedup vs baseline: 1.5306x; 1.0017x over previous
import functools

import jax
import jax.numpy as jnp
from jax import lax
from jax.experimental import pallas as pl
from jax.experimental.pallas import tpu as pltpu
from jax.experimental.pallas import tpu_sc as plsc

LANES = 16
NBUF = 5
DEPTH = 2


@functools.lru_cache(maxsize=None)
def _build(N, V, D, C):
    info = plsc.get_sparse_core_info()
    NC, NS = info.num_cores, info.num_subcores
    NW = NC * NS
    per_w = N // NW
    n_chunks = per_w // C
    n_groups = n_chunks // NBUF
    n_col = D // LANES
    mesh = plsc.VectorSubcoreMesh(core_axis_name="c", subcore_axis_name="s")

    scratch = (
        [pltpu.VMEM((per_w,), jnp.int32)]
        + [pltpu.VMEM((C,), jnp.float32) for _ in range(NBUF)]
        + [pltpu.VMEM((C, D), jnp.float32) for _ in range(NBUF)]
        + [pltpu.VMEM((D,), jnp.float32), pltpu.VMEM((D,), jnp.float32)]
        + [pltpu.VMEM_SHARED((V, D), jnp.float32)]
        + [pltpu.SemaphoreType.DMA for _ in range(3 * NBUF)]
    )

    @functools.partial(
        pl.kernel,
        mesh=mesh,
        out_type=jax.ShapeDtypeStruct((N, D), jnp.float32),
        compiler_params=pltpu.CompilerParams(needs_layout_passes=False),
        scratch_types=scratch,
    )
    def k(x_hbm, y_hbm, emb_hbm, w_hbm, b_hbm, out_hbm, *s):
        idx_all = s[0]
        yv = s[1:1 + NBUF]
        rows = s[1 + NBUF:1 + 2 * NBUF]
        w_v, b_v = s[1 + 2 * NBUF], s[2 + 2 * NBUF]
        emb_sp = s[3 + 2 * NBUF]
        gsem = s[4 + 2 * NBUF:4 + 3 * NBUF]
        osem = s[4 + 3 * NBUF:4 + 4 * NBUF]
        ysem = s[4 + 4 * NBUF:4 + 5 * NBUF]

        sid = lax.axis_index("s")
        wid = sid * NC + lax.axis_index("c")
        w0 = wid * per_w

        @pl.when(sid == 0)
        def _():
            pltpu.sync_copy(emb_hbm, emb_sp)

        pltpu.sync_copy(x_hbm.at[pl.ds(w0, per_w)], idx_all)
        pltpu.sync_copy(w_hbm, w_v)
        pltpu.sync_copy(b_hbm, b_v)
        w_regs = [w_v[pl.ds(j * LANES, LANES)] for j in range(n_col)]
        b_regs = [b_v[pl.ds(j * LANES, LANES)] for j in range(n_col)]
        plsc.subcore_barrier()

        def gdesc(ci, b):
            return pltpu.make_async_copy(
                emb_sp.at[idx_all.at[pl.ds(ci * C, C)]], rows[b], gsem[b]
            )

        def ydesc(ci, b):
            return pltpu.make_async_copy(
                y_hbm.at[pl.ds(w0 + ci * C, C)], yv[b], ysem[b]
            )

        def odesc(ci, b):
            return pltpu.make_async_copy(
                rows[b], out_hbm.at[pl.ds(w0 + ci * C, C)], osem[b]
            )

        for b in range(DEPTH):
            ydesc(b, b).start()
            gdesc(b, b).start()

        def group(g, carry):
            for b in range(NBUF):
                ci = g * NBUF + b
                p = ci + DEPTH
                pb = (b + DEPTH) % NBUF

                @pl.when((p >= NBUF) & (p < n_chunks))
                def _():
                    odesc(p - NBUF, pb).wait()

                @pl.when(p < n_chunks)
                def _():
                    ydesc(p, pb).start()
                    gdesc(p, pb).start()

                gdesc(ci, b).wait()
                ydesc(ci, b).wait()

                def blk_body(r16, acc):
                    r0 = r16 * LANES
                    y16 = yv[b][pl.ds(r0, LANES)]
                    for kk in range(LANES):
                        ysplat = jnp.take_along_axis(
                            y16,
                            jnp.full((LANES,), kk, jnp.int32),
                            axis=0,
                            mode="promise_in_bounds",
                        )
                        for j in range(n_col):
                            plsc.addupdate(
                                rows[b].at[r0 + kk, pl.ds(j * LANES, LANES)],
                                ysplat * w_regs[j] + b_regs[j],
                            )
                    return acc

                lax.fori_loop(0, C // LANES, blk_body, 0, unroll=2)
                odesc(ci, b).start()
            return carry

        lax.fori_loop(0, n_groups, group, 0, unroll=False)
        for b in range(NBUF):
            odesc(n_chunks - NBUF + b, b).wait()

    return k


def kernel(x, y, embedding, W_sig, b_sig):
    B, L = x.shape
    V, D = embedding.shape
    N = B * L
    xf = x.reshape(N)
    yf = y.reshape(N)
    w = W_sig.reshape(D)
    out = _build(N, V, D, 128)(xf, yf, embedding, w, b_sig)
    return out.reshape(B, L, D)
